# baseline (speedup 1.0000x reference)
import jax
import jax.numpy as jnp
from jax import lax
from jax.experimental import pallas as pl
from jax.experimental.pallas import tpu as pltpu
from jax.experimental.pallas import tpu_sc as plsc

_NUM_EMBS = 1000000
_EMSIZE = 64
_T, _B = 512, 1024
_N = _T * _B

_NC = 2
_NS = 16
_NW = _NC * _NS
_ROWS_W = _T // _NW


def _body(idx_hbm, emb_hbm, out_hbm, idx_v, idx_s, sem):
    wid = lax.axis_index("s") * _NC + lax.axis_index("c")
    t0 = wid * _ROWS_W

    def row(g, carry):
        t = t0 + g
        pltpu.sync_copy(idx_hbm.at[t], idx_v)

        def issue(j16, c):
            v = idx_v[pl.ds(j16 * 16, 16)]
            for l in range(16):
                pltpu.make_async_copy(
                    emb_hbm.at[pl.ds(v[l], 1)],
                    out_hbm.at[t, pl.ds(j16 * 16 + l, 1)],
                    sem).start()
            return c

        lax.fori_loop(0, _B // 16, issue, 0)
        return carry

    lax.fori_loop(0, _ROWS_W, row, 0)

    def drain(j, c):
        pltpu.make_async_copy(
            emb_hbm.at[pl.ds(0, 1)],
            out_hbm.at[t0, pl.ds(0, 1)],
            sem).wait()
        return c

    lax.fori_loop(0, _ROWS_W * _B, drain, 0, unroll=8)


def _run(idx, embedding):
    mesh = plsc.VectorSubcoreMesh(core_axis_name="c", subcore_axis_name="s")
    return pl.kernel(
        _body,
        mesh=mesh,
        out_type=jax.ShapeDtypeStruct((_T, _B, _EMSIZE), jnp.float32),
        scratch_types=[
            pltpu.VMEM((_B,), jnp.int32),
            pltpu.SMEM((_B,), jnp.int32),
            pltpu.SemaphoreType.DMA,
        ],
    )(idx, embedding)


def kernel(x, embedding, single_eval_pos):
    xs = x[..., 0]
    bad = jnp.isnan(xs) | jnp.isinf(xs)
    idx = jnp.clip(xs, 0.0, float(_NUM_EMBS - 2)).astype(jnp.int32)
    idx = jnp.where(bad, _NUM_EMBS - 1, idx)
    return _run(idx, embedding)


# speedup vs baseline: 1.3969x; 1.3969x over previous
import jax
import jax.numpy as jnp
from jax import lax
from jax.experimental import pallas as pl
from jax.experimental.pallas import tpu as pltpu
from jax.experimental.pallas import tpu_sc as plsc

_NUM_EMBS = 1000000
_EMSIZE = 64
_T, _B = 512, 1024
_N = _T * _B

_NC = 2
_NS = 16
_NW = _NC * _NS
_ROWS_W = _T // _NW
_C = 512
_NBUF = 2


def _body(idx_hbm, emb_hbm, out_hbm, idx_v,
          rows0, rows1, gsem0, gsem1, osem0, osem1):
    wid = lax.axis_index("s") * _NC + lax.axis_index("c")
    t0 = wid * _ROWS_W
    rows = (rows0, rows1)
    gsem = (gsem0, gsem1)
    osem = (osem0, osem1)

    pltpu.sync_copy(idx_hbm.at[pl.ds(t0, _ROWS_W)], idx_v)

    def gather(g, b):
        return pltpu.make_async_copy(
            emb_hbm.at[idx_v.at[g, pl.ds(b * _C, _C)]], rows[b], gsem[b])

    def out_copy(g, b):
        return pltpu.make_async_copy(
            rows[b],
            out_hbm.at[t0 + g, pl.ds(b * _C, _C), pl.ds(0, _EMSIZE)],
            osem[b])

    gather(0, 0).start()
    gather(0, 1).start()

    def group(g, carry):
        for b in range(_NBUF):
            gather(g, b).wait()
            od = out_copy(g, b)
            od.start()
            od.wait()

            @pl.when(g + 1 < _ROWS_W)
            def _():
                gather(g + 1, b).start()
        return carry

    lax.fori_loop(0, _ROWS_W, group, 0)


_BI = 400
_NBLK = _NUM_EMBS // _BI
_SBLK_W = -(-_NBLK // _NW)
_EG = _EMSIZE // 16
_STEPS = _SBLK_W * _EG


def _tbody(embT_hbm, out_hbm, eb0, eb1, tb0, tb1,
           isem0, isem1, osem0, osem1):
    wid = lax.axis_index("s") * _NC + lax.axis_index("c")
    ebuf = (eb0, eb1)
    tbuf = (tb0, tb1)
    isem = (isem0, isem1)
    osem = (osem0, osem1)

    def blk_of(s):
        return wid + _NW * (s // _EG)

    def in_dma(s, b):
        g = s % _EG
        return pltpu.make_async_copy(
            embT_hbm.at[pl.ds(g * 16, 16), pl.ds(blk_of(s) * _BI, _BI)],
            ebuf[b], isem[b])

    def out_dma(j, b):
        return pltpu.make_async_copy(
            tbuf[b], out_hbm.at[pl.ds((wid + _NW * j) * _BI, _BI)], osem[b])

    in_dma(0, 0).start()

    def step(s, carry):
        j = s // _EG
        g = s % _EG
        b = lax.rem(s, 2)

        @pl.when(blk_of(s) < _NBLK)
        def _():
            for bb in range(2):
                @pl.when(b == bb)
                def _():
                    in_dma(s, bb).wait()

                    @pl.when((s + 1 < _STEPS) & (blk_of(s + 1) < _NBLK))
                    def _():
                        in_dma(s + 1, 1 - bb).start()

            jb = lax.rem(j, 2)
            for tb in range(2):
                @pl.when((jb == tb) & (g == 0) & (j >= 2))
                def _():
                    out_dma(j - 2, tb).wait()

            def make_cloop(bb, tb):
                def run():
                    lanes = lax.iota(jnp.int32, 16)
                    scat_cols = lanes + g * 16

                    def cbody(i, c):
                        colv = jnp.full((16,), 0, jnp.int32) + i
                        v = plsc.load_gather(ebuf[bb], [lanes, colv])
                        plsc.store_scatter(tbuf[tb], [colv, scat_cols], v)
                        return c

                    lax.fori_loop(0, _BI, cbody, 0, unroll=4)
                return run

            for bb in range(2):
                for tb in range(2):
                    pl.when((b == bb) & (jb == tb))(make_cloop(bb, tb))

            for tb in range(2):
                @pl.when((jb == tb) & (g == _EG - 1))
                def _():
                    out_dma(j, tb).start()
        return carry

    lax.fori_loop(0, _STEPS, step, 0)

    nb = (_NBLK - wid + _NW - 1) // _NW
    for k in (2, 1):
        jlast = nb - k
        for tb in range(2):
            @pl.when(lax.rem(jlast, 2) == tb)
            def _():
                out_dma(jlast, tb).wait()


def _relayout(embT):
    mesh = plsc.VectorSubcoreMesh(core_axis_name="c", subcore_axis_name="s")
    return pl.kernel(
        _tbody,
        mesh=mesh,
        compiler_params=pltpu.CompilerParams(
            use_tc_tiling_on_sc=False, needs_layout_passes=False),
        out_type=jax.ShapeDtypeStruct((_NUM_EMBS, _EMSIZE), jnp.float32),
        scratch_types=[
            pltpu.VMEM((16, _BI), jnp.float32),
            pltpu.VMEM((16, _BI), jnp.float32),
            pltpu.VMEM((_BI, _EMSIZE), jnp.float32),
            pltpu.VMEM((_BI, _EMSIZE), jnp.float32),
            pltpu.SemaphoreType.DMA,
            pltpu.SemaphoreType.DMA,
            pltpu.SemaphoreType.DMA,
            pltpu.SemaphoreType.DMA,
        ],
    )(embT)


def _run(idx, emb_lin):
    mesh = plsc.VectorSubcoreMesh(core_axis_name="c", subcore_axis_name="s")
    return pl.kernel(
        _body,
        mesh=mesh,
        compiler_params=pltpu.CompilerParams(use_tc_tiling_on_sc=False),
        out_type=jax.ShapeDtypeStruct((_T, _B, 2 * _EMSIZE), jnp.float32),
        scratch_types=[
            pltpu.VMEM((_ROWS_W, _B), jnp.int32),
            pltpu.VMEM((_C, _EMSIZE), jnp.float32),
            pltpu.VMEM((_C, _EMSIZE), jnp.float32),
            pltpu.SemaphoreType.DMA,
            pltpu.SemaphoreType.DMA,
            pltpu.SemaphoreType.DMA,
            pltpu.SemaphoreType.DMA,
        ],
    )(idx, emb_lin)


def kernel(x, embedding, single_eval_pos):
    xs = x[..., 0]
    bad = jnp.isnan(xs) | jnp.isinf(xs)
    idx = jnp.clip(xs, 0.0, float(_NUM_EMBS - 2)).astype(jnp.int32)
    idx = jnp.where(bad, _NUM_EMBS - 1, idx)
    emb_lin = _relayout(embedding.T)
    out128 = _run(idx, emb_lin)
    return out128[..., :_EMSIZE]


# speedup vs baseline: 10.7924x; 7.7258x over previous
import jax
import jax.numpy as jnp
from jax import lax
from jax.experimental import pallas as pl
from jax.experimental.pallas import tpu as pltpu
from jax.experimental.pallas import tpu_sc as plsc

_NUM_EMBS = 1000000
_EMSIZE = 64
_T, _B = 512, 1024
_N = _T * _B

_NC = 2
_NS = 16
_NW = _NC * _NS
_ROWS_W = _T // _NW
_C = 512
_NBUF = 2


def _body(idx_hbm, emb_hbm, out_hbm, idx_v,
          rows0, rows1, gsem0, gsem1, osem0, osem1):
    wid = lax.axis_index("s") * _NC + lax.axis_index("c")
    t0 = wid * _ROWS_W
    rows = (rows0, rows1)
    gsem = (gsem0, gsem1)
    osem = (osem0, osem1)

    pltpu.sync_copy(idx_hbm.at[pl.ds(t0, _ROWS_W)], idx_v)

    def gather(g, b):
        return pltpu.make_async_copy(
            emb_hbm.at[idx_v.at[g, pl.ds(b * _C, _C)]], rows[b], gsem[b])

    def out_copy(g, b):
        return pltpu.make_async_copy(
            rows[b],
            out_hbm.at[t0 + g, pl.ds(b * _C, _C), pl.ds(0, _EMSIZE)],
            osem[b])

    gather(0, 0).start()
    gather(0, 1).start()

    def group(g, carry):
        for b in range(_NBUF):
            gather(g, b).wait()
            od = out_copy(g, b)
            od.start()
            od.wait()

            @pl.when(g + 1 < _ROWS_W)
            def _():
                gather(g + 1, b).start()
        return carry

    lax.fori_loop(0, _ROWS_W, group, 0)


def _run(idx, embedding):
    mesh = plsc.VectorSubcoreMesh(core_axis_name="c", subcore_axis_name="s")
    return pl.kernel(
        _body,
        mesh=mesh,
        compiler_params=pltpu.CompilerParams(use_tc_tiling_on_sc=False),
        out_type=jax.ShapeDtypeStruct((_T, _B, 2 * _EMSIZE), jnp.float32),
        scratch_types=[
            pltpu.VMEM((_ROWS_W, _B), jnp.int32),
            pltpu.VMEM((_C, _EMSIZE), jnp.float32),
            pltpu.VMEM((_C, _EMSIZE), jnp.float32),
            pltpu.SemaphoreType.DMA,
            pltpu.SemaphoreType.DMA,
            pltpu.SemaphoreType.DMA,
            pltpu.SemaphoreType.DMA,
        ],
    )(idx, embedding)


def kernel(x, embedding, single_eval_pos):
    xs = x[..., 0]
    bad = jnp.isnan(xs) | jnp.isinf(xs)
    idx = jnp.clip(xs, 0.0, float(_NUM_EMBS - 2)).astype(jnp.int32)
    idx = jnp.where(bad, _NUM_EMBS - 1, idx)
    out128 = _run(idx, embedding)
    return out128[..., :_EMSIZE]
